# baseline (speedup 1.0000x reference)
import functools

import jax
import jax.numpy as jnp
from jax import lax
from jax.experimental import pallas as pl
from jax.experimental.pallas import tpu as pltpu
from jax.experimental.pallas import tpu_sc as plsc

_N = 10000
_E = 320000
_D = 128
_NC = 2
_NS = 16
_NW = _NC * _NS
_EP = _E // _NW
_K = 80
_NCH = _EP // _K
_NP = 10240
_RPT = _NP // _NS
_ZR = 128
_BR = 1024
_NB = (_N + _BR - 1) // _BR

_DOT = functools.partial(
    lax.dot_general,
    dimension_numbers=(((1,), (0,)), ((), ())),
    preferred_element_type=jnp.float32,
    precision=lax.Precision.HIGHEST,
)


def _fill_rows(ref, nrows, ncol16, value):
    def row(i, _):
        def col(q, _):
            ref[i, pl.ds(q * 16, 16)] = jnp.full((16,), value, jnp.float32)
            return 0
        return lax.fori_loop(0, ncol16, col, 0)
    lax.fori_loop(0, nrows, row, 0)


def _mesh():
    return plsc.VectorSubcoreMesh(core_axis_name="c", subcore_axis_name="s",
                                  num_cores=_NC, num_subcores=_NS)


@functools.lru_cache(maxsize=None)
def _make_sc_scatter(with_deg):
    out_type = [jax.ShapeDtypeStruct((_NC, _NP, _D), jnp.float32)]
    scratch = (
        [pltpu.VMEM_SHARED((_NP, _D), jnp.float32)]
        + [pltpu.VMEM((_K,), jnp.int32) for _ in range(4)]
        + [pltpu.VMEM((_K,), jnp.int32) for _ in range(4)]
        + [pltpu.VMEM((2 * _K, _D), jnp.float32)]
        + [pltpu.VMEM((_ZR, _D), jnp.float32)]
        + [pltpu.SemaphoreType.DMA] * 8
    )
    if with_deg:
        out_type.append(jax.ShapeDtypeStruct((_NC, _NP), jnp.float32))
        scratch += [
            pltpu.VMEM_SHARED((_NP,), jnp.float32),
            pltpu.VMEM((_K,), jnp.float32),
            pltpu.VMEM((_RPT,), jnp.float32),
        ]

    def body(g, srcv, dstv, *refs):
        if with_deg:
            s_out, deg_out = refs[0], refs[1]
            refs = refs[2:]
            deg_acc, ones, zb1 = refs[-3], refs[-2], refs[-1]
        else:
            s_out = refs[0]
            refs = refs[1:]
        acc = refs[0]
        src_c = refs[1:5]
        dst_c = refs[5:9]
        rows2 = refs[9]
        zbuf = refs[10]
        sem_i = refs[11:15]
        sem_g = refs[15:17]
        sem_s = refs[17:19]
        rows = (rows2.at[pl.ds(0, _K)], rows2.at[pl.ds(_K, _K)])
        c = lax.axis_index("c")
        t = lax.axis_index("s")
        wid = t * _NC + c
        rbase = t * _RPT
        ebase = wid * _EP

        def idx_fetch(j, q):
            off = ebase + j * _K
            pltpu.async_copy(srcv.at[pl.ds(off, _K)], src_c[q], sem_i[q])
            pltpu.async_copy(dstv.at[pl.ds(off, _K)], dst_c[q], sem_i[q])

        def idx_wait(q):
            pltpu.make_async_copy(srcv.at[pl.ds(0, _K)], src_c[q],
                                  sem_i[q]).wait()
            pltpu.make_async_copy(dstv.at[pl.ds(0, _K)], dst_c[q],
                                  sem_i[q]).wait()

        def gather_start(q, b):
            pltpu.async_copy(g.at[src_c[q]], rows[b], sem_g[b])

        def gather_wait(q, b):
            pltpu.make_async_copy(g.at[src_c[q]], rows[b], sem_g[b]).wait()

        def scatter_start(q, b):
            pltpu.async_copy(rows[b], acc.at[dst_c[q]], sem_s[b], add=True)
            if with_deg:
                pltpu.async_copy(ones, deg_acc.at[dst_c[q]], sem_s[b],
                                 add=True)

        def scatter_wait(q, b):
            pltpu.make_async_copy(rows[b], acc.at[dst_c[q]], sem_s[b]).wait()
            if with_deg:
                pltpu.make_async_copy(ones, deg_acc.at[dst_c[q]],
                                      sem_s[b]).wait()

        idx_fetch(0, 0)
        idx_fetch(1, 1)

        _fill_rows(zbuf, _ZR, _D // 16, 0.0)
        for k in range(_RPT // _ZR):
            pltpu.sync_copy(zbuf, acc.at[pl.ds(rbase + k * _ZR, _ZR)])
        if with_deg:
            def v1row(i, _):
                zb1[pl.ds(i * 16, 16)] = jnp.zeros((16,), jnp.float32)
                return 0
            lax.fori_loop(0, _RPT // 16, v1row, 0)
            for q in range(_K // 16):
                ones[pl.ds(q * 16, 16)] = jnp.ones((16,), jnp.float32)
            pltpu.sync_copy(zb1, deg_acc.at[pl.ds(rbase, _RPT)])

        idx_wait(0)
        gather_start(0, 0)
        plsc.subcore_barrier()

        def quad(i, _):
            for u in range(4):
                j = 4 * i + u
                b = u % 2
                bn = 1 - b
                q = u
                qn = (u + 1) % 4
                q2 = (u + 2) % 4
                @pl.when(j < _NCH)
                def _():
                    gather_wait(q, b)
                    scatter_start(q, b)
                    @pl.when(j >= 1)
                    def _():
                        scatter_wait((u + 3) % 4, bn)
                    @pl.when(j + 2 < _NCH)
                    def _():
                        idx_fetch(j + 2, q2)
                    @pl.when(j + 1 < _NCH)
                    def _():
                        idx_wait(qn)
                        gather_start(qn, bn)
            return 0
        lax.fori_loop(0, (_NCH + 3) // 4, quad, 0)
        scatter_wait((_NCH - 1) % 4, (_NCH - 1) % 2)

        plsc.subcore_barrier()
        for k in range(_RPT // _ZR):
            r = rbase + k * _ZR
            pltpu.sync_copy(acc.at[pl.ds(r, _ZR)], s_out.at[c, pl.ds(r, _ZR)])
        if with_deg:
            pltpu.sync_copy(deg_acc.at[pl.ds(rbase, _RPT)],
                            deg_out.at[c, pl.ds(rbase, _RPT)])

    return pl.kernel(body, out_type=tuple(out_type), mesh=_mesh(),
                     scratch_types=tuple(scratch))


def _mm_body(h_ref, w_ref, o_ref):
    o_ref[...] = _DOT(h_ref[...], w_ref[...])


def _inv_deg(d_ref):
    deg = jnp.sum(d_ref[...], axis=0)[:, None]
    return 1.0 / jnp.maximum(deg, 1.0)


def _step_body(h_ref, s_ref, d_ref, ws_ref, wn_ref, b_ref, h1_ref, g1_ref):
    agg = (s_ref[0] + s_ref[1]) * _inv_deg(d_ref)
    z = _DOT(h_ref[...], ws_ref[...]) + agg + b_ref[...]
    h1 = jnp.maximum(z, 0.0)
    h1_ref[...] = h1
    g1_ref[...] = _DOT(h1, wn_ref[...])


def _final_body(h_ref, s_ref, d_ref, ws_ref, b_ref, o_ref):
    agg = (s_ref[0] + s_ref[1]) * _inv_deg(d_ref)
    o_ref[...] = _DOT(h_ref[...], ws_ref[...]) + agg + b_ref[...]


_b_h = pl.BlockSpec((_BR, _D), lambda i: (i, 0))
_b_w = pl.BlockSpec((_D, _D), lambda i: (0, 0))
_b_b = pl.BlockSpec((1, _D), lambda i: (0, 0))
_b_s = pl.BlockSpec((_NC, _BR, _D), lambda i: (0, i, 0))
_b_d = pl.BlockSpec((_NC, _BR), lambda i: (0, i))
_o_h = jax.ShapeDtypeStruct((_N, _D), jnp.float32)

_mm = pl.pallas_call(
    _mm_body, grid=(_NB,),
    in_specs=[_b_h, _b_w], out_specs=_b_h, out_shape=_o_h)

_step = pl.pallas_call(
    _step_body, grid=(_NB,),
    in_specs=[_b_h, _b_s, _b_d, _b_w, _b_w, _b_b],
    out_specs=(_b_h, _b_h), out_shape=(_o_h, _o_h))

_final = pl.pallas_call(
    _final_body, grid=(_NB,),
    in_specs=[_b_h, _b_s, _b_d, _b_w, _b_b],
    out_specs=_b_h, out_shape=_o_h)


def kernel(feats, edge_index, W_self_0, W_neigh_0, b_0,
           W_self_1, W_neigh_1, b_1, W_self_2, W_neigh_2, b_2):
    src = edge_index[0]
    dst = edge_index[1]
    b0 = b_0.reshape(1, _D)
    b1 = b_1.reshape(1, _D)
    b2 = b_2.reshape(1, _D)

    g0 = _mm(feats, W_neigh_0)
    s0, deg = _make_sc_scatter(True)(g0, src, dst)
    h1, g1 = _step(feats, s0, deg, W_self_0, W_neigh_1, b0)
    (s1,) = _make_sc_scatter(False)(g1, src, dst)
    h2, g2 = _step(h1, s1, deg, W_self_1, W_neigh_2, b1)
    (s2,) = _make_sc_scatter(False)(g2, src, dst)
    h3 = _final(h2, s2, deg, W_self_2, b2)
    return h3


# speedup vs baseline: 1.2116x; 1.2116x over previous
import functools

import jax
import jax.numpy as jnp
from jax import lax
from jax.experimental import pallas as pl
from jax.experimental.pallas import tpu as pltpu
from jax.experimental.pallas import tpu_sc as plsc

_N = 10000
_E = 320000
_D = 128
_NC = 2
_NS = 16
_NW = _NC * _NS
_EP = _E // _NW
_K = 128
_NCH = _EP // _K
_KT = _EP - _NCH * _K
_NP = 10240
_RPT = _NP // _NS
_ZR = 32
_BR = 1024
_NB = (_N + _BR - 1) // _BR

_DOT = functools.partial(
    lax.dot_general,
    dimension_numbers=(((1,), (0,)), ((), ())),
    preferred_element_type=jnp.float32,
    precision=lax.Precision.HIGHEST,
)


def _fill_rows(ref, nrows, ncol16, value):
    def row(i, _):
        def col(q, _):
            ref[i, pl.ds(q * 16, 16)] = jnp.full((16,), value, jnp.float32)
            return 0
        return lax.fori_loop(0, ncol16, col, 0)
    lax.fori_loop(0, nrows, row, 0)


def _mesh():
    return plsc.VectorSubcoreMesh(core_axis_name="c", subcore_axis_name="s",
                                  num_cores=_NC, num_subcores=_NS)


@functools.lru_cache(maxsize=None)
def _make_sc_scatter(with_deg):
    out_type = [jax.ShapeDtypeStruct((_NC, _NP, _D), jnp.float32)]
    scratch = (
        [pltpu.VMEM_SHARED((_NP, _D), jnp.float32)]
        + [pltpu.VMEM((_K,), jnp.int32) for _ in range(2)]
        + [pltpu.VMEM((_K,), jnp.int32) for _ in range(2)]
        + [pltpu.VMEM((2 * _K, _D), jnp.float32)]
        + [pltpu.VMEM((_KT,), jnp.int32) for _ in range(2)]
        + [pltpu.VMEM((_KT, _D), jnp.float32)]
        + [pltpu.VMEM((_ZR, _D), jnp.float32)]
        + [pltpu.SemaphoreType.DMA] * 5
    )
    if with_deg:
        out_type.append(jax.ShapeDtypeStruct((_NC, _NP), jnp.float32))
        scratch += [
            pltpu.VMEM_SHARED((_NP,), jnp.float32),
            pltpu.VMEM((_K,), jnp.float32),
            pltpu.VMEM((_RPT,), jnp.float32),
        ]

    def body(g, srcv, dstv, *refs):
        if with_deg:
            s_out, deg_out = refs[0], refs[1]
            deg_acc, ones, zb1 = refs[-3], refs[-2], refs[-1]
            refs = refs[2:-3]
        else:
            s_out = refs[0]
            refs = refs[1:]
        (acc, src_c0, src_c1, dst_c0, dst_c1, rows2,
         src_t, dst_t, rows_t, zbuf, semi0, semi1, semg0, semg1, semt) = refs
        src_c = (src_c0, src_c1)
        dst_c = (dst_c0, dst_c1)
        rows = (rows2.at[pl.ds(0, _K)], rows2.at[pl.ds(_K, _K)])
        sem_i = (semi0, semi1)
        sem_g = (semg0, semg1)
        c = lax.axis_index("c")
        t = lax.axis_index("s")
        wid = t * _NC + c
        rbase = t * _RPT
        ebase = wid * _EP

        def idx_fetch(j, b):
            off = ebase + j * _K
            pltpu.async_copy(srcv.at[pl.ds(off, _K)], src_c[b], sem_i[b])
            pltpu.async_copy(dstv.at[pl.ds(off, _K)], dst_c[b], sem_i[b])

        def idx_wait(b):
            pltpu.make_async_copy(srcv.at[pl.ds(0, _K)], src_c[b],
                                  sem_i[b]).wait()
            pltpu.make_async_copy(dstv.at[pl.ds(0, _K)], dst_c[b],
                                  sem_i[b]).wait()

        def gather_start(b):
            pltpu.async_copy(g.at[src_c[b]], rows[b], sem_g[b])

        def gather_wait(b):
            pltpu.make_async_copy(g.at[src_c[b]], rows[b], sem_g[b]).wait()

        def scatter(b):
            pltpu.sync_copy(rows[b], acc.at[dst_c[b]], add=True)
            if with_deg:
                pltpu.sync_copy(ones, deg_acc.at[dst_c[b]], add=True)

        idx_fetch(0, 0)
        idx_fetch(1, 1)
        toff = ebase + _NCH * _K
        pltpu.async_copy(srcv.at[pl.ds(toff, _KT)], src_t, semt)
        pltpu.async_copy(dstv.at[pl.ds(toff, _KT)], dst_t, semt)

        _fill_rows(zbuf, _ZR, _D // 16, 0.0)
        for k in range(_RPT // _ZR):
            pltpu.sync_copy(zbuf, acc.at[pl.ds(rbase + k * _ZR, _ZR)])
        if with_deg:
            def v1row(i, _):
                zb1[pl.ds(i * 16, 16)] = jnp.zeros((16,), jnp.float32)
                return 0
            lax.fori_loop(0, _RPT // 16, v1row, 0)
            for q in range(_K // 16):
                ones[pl.ds(q * 16, 16)] = jnp.ones((16,), jnp.float32)
            pltpu.sync_copy(zb1, deg_acc.at[pl.ds(rbase, _RPT)])

        idx_wait(0)
        gather_start(0)
        plsc.subcore_barrier()

        def pair(i, _):
            for b in range(2):
                j = 2 * i + b
                bn = 1 - b
                @pl.when(j + 1 < _NCH)
                def _():
                    idx_wait(bn)
                    gather_start(bn)
                gather_wait(b)
                scatter(b)
                @pl.when(j + 2 < _NCH)
                def _():
                    idx_fetch(j + 2, b)
            return 0
        lax.fori_loop(0, _NCH // 2, pair, 0)

        pltpu.make_async_copy(srcv.at[pl.ds(0, _KT)], src_t, semt).wait()
        pltpu.make_async_copy(dstv.at[pl.ds(0, _KT)], dst_t, semt).wait()
        pltpu.async_copy(g.at[src_t], rows_t, semt).wait()
        pltpu.sync_copy(rows_t, acc.at[dst_t], add=True)
        if with_deg:
            pltpu.sync_copy(ones.at[pl.ds(0, _KT)], deg_acc.at[dst_t],
                            add=True)

        plsc.subcore_barrier()
        for k in range(_RPT // 128):
            r = rbase + k * 128
            pltpu.sync_copy(acc.at[pl.ds(r, 128)], s_out.at[c, pl.ds(r, 128)])
        if with_deg:
            pltpu.sync_copy(deg_acc.at[pl.ds(rbase, _RPT)],
                            deg_out.at[c, pl.ds(rbase, _RPT)])

    return pl.kernel(body, out_type=tuple(out_type), mesh=_mesh(),
                     scratch_types=tuple(scratch))


def _mm_body(h_ref, w_ref, o_ref):
    o_ref[...] = _DOT(h_ref[...], w_ref[...])


def _inv_deg(d_ref):
    deg = jnp.sum(d_ref[...], axis=0)[:, None]
    return 1.0 / jnp.maximum(deg, 1.0)


def _step_body(h_ref, s_ref, d_ref, ws_ref, wn_ref, b_ref, h1_ref, g1_ref):
    agg = (s_ref[0] + s_ref[1]) * _inv_deg(d_ref)
    z = _DOT(h_ref[...], ws_ref[...]) + agg + b_ref[...]
    h1 = jnp.maximum(z, 0.0)
    h1_ref[...] = h1
    g1_ref[...] = _DOT(h1, wn_ref[...])


def _final_body(h_ref, s_ref, d_ref, ws_ref, b_ref, o_ref):
    agg = (s_ref[0] + s_ref[1]) * _inv_deg(d_ref)
    o_ref[...] = _DOT(h_ref[...], ws_ref[...]) + agg + b_ref[...]


_b_h = pl.BlockSpec((_BR, _D), lambda i: (i, 0))
_b_w = pl.BlockSpec((_D, _D), lambda i: (0, 0))
_b_b = pl.BlockSpec((1, _D), lambda i: (0, 0))
_b_s = pl.BlockSpec((_NC, _BR, _D), lambda i: (0, i, 0))
_b_d = pl.BlockSpec((_NC, _BR), lambda i: (0, i))
_o_h = jax.ShapeDtypeStruct((_N, _D), jnp.float32)

_mm = pl.pallas_call(
    _mm_body, grid=(_NB,),
    in_specs=[_b_h, _b_w], out_specs=_b_h, out_shape=_o_h)

_step = pl.pallas_call(
    _step_body, grid=(_NB,),
    in_specs=[_b_h, _b_s, _b_d, _b_w, _b_w, _b_b],
    out_specs=(_b_h, _b_h), out_shape=(_o_h, _o_h))

_final = pl.pallas_call(
    _final_body, grid=(_NB,),
    in_specs=[_b_h, _b_s, _b_d, _b_w, _b_b],
    out_specs=_b_h, out_shape=_o_h)


def kernel(feats, edge_index, W_self_0, W_neigh_0, b_0,
           W_self_1, W_neigh_1, b_1, W_self_2, W_neigh_2, b_2):
    src = edge_index[0]
    dst = edge_index[1]
    b0 = b_0.reshape(1, _D)
    b1 = b_1.reshape(1, _D)
    b2 = b_2.reshape(1, _D)

    g0 = _mm(feats, W_neigh_0)
    s0, deg = _make_sc_scatter(True)(g0, src, dst)
    h1, g1 = _step(feats, s0, deg, W_self_0, W_neigh_1, b0)
    (s1,) = _make_sc_scatter(False)(g1, src, dst)
    h2, g2 = _step(h1, s1, deg, W_self_1, W_neigh_2, b1)
    (s2,) = _make_sc_scatter(False)(g2, src, dst)
    h3 = _final(h2, s2, deg, W_self_2, b2)
    return h3
